# baseline (speedup 1.0000x reference)
import functools

import jax
import jax.numpy as jnp
from jax.experimental import pallas as pl
from jax.experimental.pallas import tpu as pltpu

P_GEN = 0.2
P_MEM = 0.1
BASE_KEY_SEED = 12345

_ROTATIONS = ((13, 15, 26, 6), (17, 29, 16, 24))


def _threefry2x32(k1, k2, x1, x2):
    ks0 = k1
    ks1 = k2
    ks2 = k1 ^ k2 ^ jnp.uint32(0x1BD11BDA)
    ks = (ks0, ks1, ks2)
    a = x1 + ks0
    b = x2 + ks1
    for i in range(5):
        for r in _ROTATIONS[i % 2]:
            a = a + b
            b = (b << jnp.uint32(r)) | (b >> jnp.uint32(32 - r))
            b = a ^ b
        a = a + ks[(i + 1) % 3]
        b = b + ks[(i + 2) % 3] + jnp.uint32(i + 1)
    return a, b


def _mask_table(idx_u32, n_channels, fixed_channels):
    bsz = idx_u32.shape[0]
    zero = jnp.zeros_like(idx_u32)
    fk1, fk2 = _threefry2x32(
        jnp.uint32(0), jnp.uint32(BASE_KEY_SEED), zero, idx_u32
    )
    c = jax.lax.broadcasted_iota(jnp.int32, (bsz, n_channels), 1)
    j = (c - fixed_channels).astype(jnp.uint32)
    o1, o2 = _threefry2x32(fk1, fk2, jnp.zeros_like(j), j)
    bits = o1 ^ o2
    fbits = (bits >> jnp.uint32(9)) | jnp.uint32(0x3F800000)
    u = jax.lax.bitcast_convert_type(fbits, jnp.float32) - jnp.float32(1.0)
    keep = (u < jnp.float32(P_MEM)).astype(jnp.float32)
    return jnp.where(c < fixed_channels, jnp.float32(1.0), keep)


def _tied_dropout_kernel(idx_ref, x_ref, o_ref, *, fixed_channels):
    n_channels = x_ref.shape[2]
    mask = _mask_table(
        idx_ref[...].astype(jnp.uint32), n_channels, fixed_channels
    )
    o_ref[...] = x_ref[...] * mask[None, :, :]


@jax.jit
def kernel(X, indices):
    B, C, H, W = X.shape
    fixed_channels = int(P_GEN * C)
    hw = H * W
    xt = jnp.transpose(X, (2, 3, 0, 1)).reshape(hw, B, C)
    idx2 = indices.astype(jnp.int32).reshape(B, 1)

    BB = 8
    out = pl.pallas_call(
        functools.partial(_tied_dropout_kernel, fixed_channels=fixed_channels),
        grid=(B // BB,),
        in_specs=[
            pl.BlockSpec((BB, 1), lambda s: (s, 0)),
            pl.BlockSpec((hw, BB, C), lambda s: (0, s, 0)),
        ],
        out_specs=pl.BlockSpec((hw, BB, C), lambda s: (0, s, 0)),
        out_shape=jax.ShapeDtypeStruct((hw, B, C), X.dtype),
    )(idx2, xt)
    return jnp.transpose(out.reshape(H, W, B, C), (2, 3, 0, 1))


# speedup vs baseline: 1.0347x; 1.0347x over previous
import functools

import jax
import jax.numpy as jnp
from jax.experimental import pallas as pl
from jax.experimental.pallas import tpu as pltpu

P_GEN = 0.2
P_MEM = 0.1
BASE_KEY_SEED = 12345

_ROTATIONS = ((13, 15, 26, 6), (17, 29, 16, 24))


def _threefry2x32(k1, k2, x1, x2):
    ks0 = k1
    ks1 = k2
    ks2 = k1 ^ k2 ^ jnp.uint32(0x1BD11BDA)
    ks = (ks0, ks1, ks2)
    a = x1 + ks0
    b = x2 + ks1
    for i in range(5):
        for r in _ROTATIONS[i % 2]:
            a = a + b
            b = (b << jnp.uint32(r)) | (b >> jnp.uint32(32 - r))
            b = a ^ b
        a = a + ks[(i + 1) % 3]
        b = b + ks[(i + 2) % 3] + jnp.uint32(i + 1)
    return a, b


def _mask_table(idx_u32, n_channels, fixed_channels):
    bsz = idx_u32.shape[0]
    zero = jnp.zeros_like(idx_u32)
    fk1, fk2 = _threefry2x32(
        jnp.uint32(0), jnp.uint32(BASE_KEY_SEED), zero, idx_u32
    )
    c = jax.lax.broadcasted_iota(jnp.int32, (bsz, n_channels), 1)
    j = (c - fixed_channels).astype(jnp.uint32)
    o1, o2 = _threefry2x32(fk1, fk2, jnp.zeros_like(j), j)
    bits = o1 ^ o2
    fbits = (bits >> jnp.uint32(9)) | jnp.uint32(0x3F800000)
    u = jax.lax.bitcast_convert_type(fbits, jnp.float32) - jnp.float32(1.0)
    keep = (u < jnp.float32(P_MEM)).astype(jnp.float32)
    return jnp.where(c < fixed_channels, jnp.float32(1.0), keep)


def _tied_dropout_kernel(idx_ref, x_ref, o_ref, *, fixed_channels):
    n_channels = x_ref.shape[2]
    mask = _mask_table(
        idx_ref[...].astype(jnp.uint32), n_channels, fixed_channels
    )
    o_ref[...] = x_ref[...] * mask[None, :, :]


@jax.jit
def kernel(X, indices):
    B, C, H, W = X.shape
    fixed_channels = int(P_GEN * C)
    hw = H * W
    xt = jnp.transpose(X, (2, 3, 0, 1)).reshape(hw, B, C)
    idx2 = indices.astype(jnp.int32).reshape(B, 1)

    BB = 128
    GG = 49
    out = pl.pallas_call(
        functools.partial(_tied_dropout_kernel, fixed_channels=fixed_channels),
        grid=(hw // GG, B // BB),
        in_specs=[
            pl.BlockSpec((BB, 1), lambda h, s: (s, 0)),
            pl.BlockSpec((GG, BB, C), lambda h, s: (h, s, 0)),
        ],
        out_specs=pl.BlockSpec((GG, BB, C), lambda h, s: (h, s, 0)),
        out_shape=jax.ShapeDtypeStruct((hw, B, C), X.dtype),
    )(idx2, xt)
    return jnp.transpose(out.reshape(H, W, B, C), (2, 3, 0, 1))


# speedup vs baseline: 1.0674x; 1.0316x over previous
import functools

import jax
import jax.numpy as jnp
from jax.experimental import pallas as pl
from jax.experimental.pallas import tpu as pltpu

P_GEN = 0.2
P_MEM = 0.1
BASE_KEY_SEED = 12345

_ROTATIONS = ((13, 15, 26, 6), (17, 29, 16, 24))


def _threefry2x32(k1, k2, x1, x2):
    ks0 = k1
    ks1 = k2
    ks2 = k1 ^ k2 ^ jnp.uint32(0x1BD11BDA)
    ks = (ks0, ks1, ks2)
    a = x1 + ks0
    b = x2 + ks1
    for i in range(5):
        for r in _ROTATIONS[i % 2]:
            a = a + b
            b = (b << jnp.uint32(r)) | (b >> jnp.uint32(32 - r))
            b = a ^ b
        a = a + ks[(i + 1) % 3]
        b = b + ks[(i + 2) % 3] + jnp.uint32(i + 1)
    return a, b


def _mask_table(idx_u32, n_channels, fixed_channels):
    bsz = idx_u32.shape[0]
    zero = jnp.zeros_like(idx_u32)
    fk1, fk2 = _threefry2x32(
        jnp.uint32(0), jnp.uint32(BASE_KEY_SEED), zero, idx_u32
    )
    c = jax.lax.broadcasted_iota(jnp.int32, (bsz, n_channels), 1)
    j = (c - fixed_channels).astype(jnp.uint32)
    o1, o2 = _threefry2x32(fk1, fk2, jnp.zeros_like(j), j)
    bits = o1 ^ o2
    fbits = (bits >> jnp.uint32(9)) | jnp.uint32(0x3F800000)
    u = jax.lax.bitcast_convert_type(fbits, jnp.float32) - jnp.float32(1.0)
    keep = (u < jnp.float32(P_MEM)).astype(jnp.float32)
    return jnp.where(c < fixed_channels, jnp.float32(1.0), keep)


def _tied_dropout_kernel(idx_ref, x_ref, o_ref, *, fixed_channels):
    n_channels = x_ref.shape[2]
    mask = _mask_table(
        idx_ref[...].astype(jnp.uint32), n_channels, fixed_channels
    )
    o_ref[...] = x_ref[...] * mask[None, :, :]


@jax.jit
def kernel(X, indices):
    B, C, H, W = X.shape
    fixed_channels = int(P_GEN * C)
    hw = H * W
    xt = jnp.transpose(X, (2, 3, 0, 1)).reshape(hw, B, C)
    idx2 = indices.astype(jnp.int32).reshape(B, 1)

    BB = 64
    GG = 98
    out = pl.pallas_call(
        functools.partial(_tied_dropout_kernel, fixed_channels=fixed_channels),
        grid=(hw // GG, B // BB),
        in_specs=[
            pl.BlockSpec((BB, 1), lambda h, s: (s, 0)),
            pl.BlockSpec((GG, BB, C), lambda h, s: (h, s, 0)),
        ],
        out_specs=pl.BlockSpec((GG, BB, C), lambda h, s: (h, s, 0)),
        out_shape=jax.ShapeDtypeStruct((hw, B, C), X.dtype),
    )(idx2, xt)
    return jnp.transpose(out.reshape(H, W, B, C), (2, 3, 0, 1))


# speedup vs baseline: 1.0696x; 1.0020x over previous
import functools

import jax
import jax.numpy as jnp
from jax.experimental import pallas as pl
from jax.experimental.pallas import tpu as pltpu

P_GEN = 0.2
P_MEM = 0.1
BASE_KEY_SEED = 12345

_ROTATIONS = ((13, 15, 26, 6), (17, 29, 16, 24))


def _threefry2x32(k1, k2, x1, x2):
    ks0 = k1
    ks1 = k2
    ks2 = k1 ^ k2 ^ jnp.uint32(0x1BD11BDA)
    ks = (ks0, ks1, ks2)
    a = x1 + ks0
    b = x2 + ks1
    for i in range(5):
        for r in _ROTATIONS[i % 2]:
            a = a + b
            b = (b << jnp.uint32(r)) | (b >> jnp.uint32(32 - r))
            b = a ^ b
        a = a + ks[(i + 1) % 3]
        b = b + ks[(i + 2) % 3] + jnp.uint32(i + 1)
    return a, b


def _mask_table(idx_u32, n_channels, fixed_channels):
    bsz = idx_u32.shape[0]
    zero = jnp.zeros_like(idx_u32)
    fk1, fk2 = _threefry2x32(
        jnp.uint32(0), jnp.uint32(BASE_KEY_SEED), zero, idx_u32
    )
    c = jax.lax.broadcasted_iota(jnp.int32, (bsz, n_channels), 1)
    j = (c - fixed_channels).astype(jnp.uint32)
    o1, o2 = _threefry2x32(fk1, fk2, jnp.zeros_like(j), j)
    bits = o1 ^ o2
    keep = ((bits >> jnp.uint32(9)) < jnp.uint32(838861)).astype(jnp.float32)
    return jnp.where(c < fixed_channels, jnp.float32(1.0), keep)


def _tied_dropout_kernel(idx_ref, x_ref, o_ref, *, fixed_channels):
    n_channels = x_ref.shape[2]
    mask = _mask_table(
        idx_ref[...].astype(jnp.uint32), n_channels, fixed_channels
    )
    o_ref[...] = x_ref[...] * mask[None, :, :]


@jax.jit
def kernel(X, indices):
    B, C, H, W = X.shape
    fixed_channels = int(P_GEN * C)
    hw = H * W
    xt = jnp.transpose(X, (2, 3, 0, 1)).reshape(hw, B, C)
    idx2 = indices.astype(jnp.int32).reshape(B, 1)

    BB = 32
    out = pl.pallas_call(
        functools.partial(_tied_dropout_kernel, fixed_channels=fixed_channels),
        grid=(B // BB,),
        in_specs=[
            pl.BlockSpec((BB, 1), lambda s: (s, 0)),
            pl.BlockSpec((hw, BB, C), lambda s: (0, s, 0)),
        ],
        out_specs=pl.BlockSpec((hw, BB, C), lambda s: (0, s, 0)),
        out_shape=jax.ShapeDtypeStruct((hw, B, C), X.dtype),
    )(idx2, xt)
    return jnp.transpose(out.reshape(H, W, B, C), (2, 3, 0, 1))
